# baseline (speedup 1.0000x reference)
import functools

import jax
import jax.numpy as jnp
from jax import lax
from jax.experimental import pallas as pl
from jax.experimental.pallas import tpu as pltpu
from jax.experimental.pallas import tpu_sc as plsc

N = 1_000_000
D = 64
B = 98304
TWOB = 2 * B

NC = 2
NS = 16
NW = NC * NS
PER_W = TWOB // NW
CHUNK = 128
NCHUNK = PER_W // CHUNK

BLK = 2048
NBLK = B // BLK


def _sc_gather_fn():
    mesh = plsc.VectorSubcoreMesh(core_axis_name="c", subcore_axis_name="s")

    @functools.partial(
        pl.kernel,
        out_type=jax.ShapeDtypeStruct((TWOB, D), jnp.bfloat16),
        mesh=mesh,
        compiler_params=pltpu.CompilerParams(use_tc_tiling_on_sc=False),
        scratch_types=[
            pltpu.VMEM((NCHUNK, CHUNK), jnp.int32),
            pltpu.VMEM((CHUNK, D), jnp.bfloat16),
            pltpu.VMEM((CHUNK, D), jnp.bfloat16),
            pltpu.SemaphoreType.DMA,
            pltpu.SemaphoreType.DMA,
        ],
    )
    def sc_gather(u_hbm, table_hbm, out_hbm, idx_v, rows_a, rows_b, sem_a, sem_b):
        wid = lax.axis_index("s") * NC + lax.axis_index("c")
        rowbase = wid * PER_W
        pltpu.sync_copy(u_hbm.at[pl.ds(wid * NCHUNK, NCHUNK)], idx_v)

        def step(i, _):
            c0 = 2 * i
            cp_a = pltpu.async_copy(table_hbm.at[idx_v.at[c0]], rows_a, sem_a)
            cp_b = pltpu.async_copy(table_hbm.at[idx_v.at[c0 + 1]], rows_b, sem_b)
            cp_a.wait()
            pltpu.sync_copy(rows_a, out_hbm.at[pl.ds(rowbase + c0 * CHUNK, CHUNK)])
            cp_b.wait()
            pltpu.sync_copy(rows_b, out_hbm.at[pl.ds(rowbase + (c0 + 1) * CHUNK, CHUNK)])
            return 0

        lax.fori_loop(0, NCHUNK // 2, step, 0)

    return sc_gather


def _tc_loss_body(ei_ref, ej_ref, lab_ref, lay_ref, w_ref, le_ref, acc_ref):
    x = jnp.dot(ei_ref[...], w_ref[...], preferred_element_type=jnp.float32)
    y = jnp.dot(ej_ref[...], w_ref[...], preferred_element_type=jnp.float32)
    lay = lay_ref[...]
    onehot = (lay == lax.broadcasted_iota(jnp.int32, (BLK, 8), 1)).astype(jnp.float32)
    l = jnp.dot(onehot, le_ref[...], preferred_element_type=jnp.float32)
    ri = x + l
    rj = y + l
    inner = jnp.sum(ri * rj, axis=1, keepdims=True)
    t = lab_ref[...] * inner
    part = jnp.sum(jax.nn.log_sigmoid(t))

    @pl.when(pl.program_id(0) == 0)
    def _():
        acc_ref[0, 0] = 0.0

    acc_ref[0, 0] += -part


def kernel(u_i, u_j, this_layer, label, embedding, L_embedding, W):
    table16 = embedding.astype(jnp.bfloat16)
    u_all = jnp.concatenate([u_i, u_j]).astype(jnp.int32).reshape(TWOB // CHUNK, CHUNK)
    gathered = _sc_gather_fn()(u_all, table16)
    lab2 = label.astype(jnp.float32).reshape(B, 1)
    lay2 = this_layer.astype(jnp.int32).reshape(B, 1)
    le_pad = jnp.zeros((8, D), jnp.float32).at[:5].set(L_embedding)
    w16 = W.astype(jnp.bfloat16)

    loss = pl.pallas_call(
        _tc_loss_body,
        grid=(NBLK,),
        in_specs=[
            pl.BlockSpec((BLK, D), lambda i: (i, 0)),
            pl.BlockSpec((BLK, D), lambda i: (NBLK + i, 0)),
            pl.BlockSpec((BLK, 1), lambda i: (i, 0)),
            pl.BlockSpec((BLK, 1), lambda i: (i, 0)),
            pl.BlockSpec((D, D), lambda i: (0, 0)),
            pl.BlockSpec((8, D), lambda i: (0, 0)),
        ],
        out_specs=pl.BlockSpec(memory_space=pltpu.SMEM),
        out_shape=jax.ShapeDtypeStruct((1, 1), jnp.float32),
    )(gathered, gathered, lab2, lay2, w16, le_pad)
    return loss[0, 0]


# speedup vs baseline: 1.0115x; 1.0115x over previous
import functools

import jax
import jax.numpy as jnp
from jax import lax
from jax.experimental import pallas as pl
from jax.experimental.pallas import tpu as pltpu
from jax.experimental.pallas import tpu_sc as plsc

N = 1_000_000
D = 64
B = 98304
TWOB = 2 * B
NLAYER = 5

NC = 2
NS = 16
NW = NC * NS
PER_W = TWOB // NW
CHUNK = 128
NCHUNK = PER_W // CHUNK

BLK = 2048
NBLK = B // BLK


def _sc_gather_fn():
    mesh = plsc.VectorSubcoreMesh(core_axis_name="c", subcore_axis_name="s")

    @functools.partial(
        pl.kernel,
        out_type=jax.ShapeDtypeStruct((TWOB, D), jnp.float32),
        mesh=mesh,
        compiler_params=pltpu.CompilerParams(use_tc_tiling_on_sc=False),
        scratch_types=[
            pltpu.VMEM((NCHUNK, CHUNK), jnp.int32),
            pltpu.VMEM((CHUNK, D), jnp.float32),
            pltpu.VMEM((CHUNK, D), jnp.float32),
            pltpu.SemaphoreType.DMA,
            pltpu.SemaphoreType.DMA,
        ],
    )
    def sc_gather(u_hbm, table_hbm, out_hbm, idx_v, rows_a, rows_b, sem_a, sem_b):
        wid = lax.axis_index("s") * NC + lax.axis_index("c")
        rowbase = wid * PER_W
        pltpu.sync_copy(u_hbm.at[pl.ds(wid * NCHUNK, NCHUNK)], idx_v)

        def step(i, _):
            c0 = 2 * i
            cp_a = pltpu.async_copy(table_hbm.at[idx_v.at[c0]], rows_a, sem_a)
            cp_b = pltpu.async_copy(table_hbm.at[idx_v.at[c0 + 1]], rows_b, sem_b)
            cp_a.wait()
            pltpu.sync_copy(rows_a, out_hbm.at[pl.ds(rowbase + c0 * CHUNK, CHUNK)])
            cp_b.wait()
            pltpu.sync_copy(rows_b, out_hbm.at[pl.ds(rowbase + (c0 + 1) * CHUNK, CHUNK)])
            return 0

        lax.fori_loop(0, NCHUNK // 2, step, 0)

    return sc_gather


def _tc_loss_body(g_ref, lab_ref, lay_ref, w_ref, lt_ref, q_ref, acc_ref):
    blk = g_ref[...]
    ei = blk[:, :D]
    ej = blk[:, D:]
    x = jnp.dot(ei, w_ref[...], preferred_element_type=jnp.float32)
    y = jnp.dot(ej, w_ref[...], preferred_element_type=jnp.float32)
    rxy = jnp.sum(x * y, axis=1)
    s1 = jnp.dot(x + y, lt_ref[...], preferred_element_type=jnp.float32)
    lay = lay_ref[...]
    inner = rxy
    for k in range(NLAYER):
        mk = (lay == k).astype(jnp.float32)
        inner = inner + mk * (s1[:, k] + q_ref[0, k])
    t = lab_ref[...] * inner
    part = jnp.sum(jax.nn.log_sigmoid(t))

    @pl.when(pl.program_id(0) == 0)
    def _():
        acc_ref[0, 0] = 0.0

    acc_ref[0, 0] += -part


def kernel(u_i, u_j, this_layer, label, embedding, L_embedding, W):
    m = lax.iota(jnp.int32, TWOB)
    u_all = jnp.where(
        m % 2 == 0,
        jnp.repeat(u_i.astype(jnp.int32), 2),
        jnp.repeat(u_j.astype(jnp.int32), 2),
    ).reshape(TWOB // CHUNK, CHUNK)
    gathered = _sc_gather_fn()(u_all, embedding)
    g2 = gathered.reshape(B, 2 * D)

    lab = label.astype(jnp.float32)
    lay = this_layer.astype(jnp.int32)
    lt = jnp.zeros((D, 8), jnp.float32).at[:, :NLAYER].set(L_embedding.T)
    q = jnp.zeros((1, 8), jnp.float32).at[0, :NLAYER].set(
        jnp.sum(L_embedding * L_embedding, axis=1))

    loss = pl.pallas_call(
        _tc_loss_body,
        grid=(NBLK,),
        in_specs=[
            pl.BlockSpec((BLK, 2 * D), lambda i: (i, 0)),
            pl.BlockSpec((BLK,), lambda i: (i,)),
            pl.BlockSpec((BLK,), lambda i: (i,)),
            pl.BlockSpec((D, D), lambda i: (0, 0)),
            pl.BlockSpec((D, 8), lambda i: (0, 0)),
            pl.BlockSpec((1, 8), lambda i: (0, 0)),
        ],
        out_specs=pl.BlockSpec(memory_space=pltpu.SMEM),
        out_shape=jax.ShapeDtypeStruct((1, 1), jnp.float32),
    )(g2, lab, lay, W, lt, q)
    return loss[0, 0]


# speedup vs baseline: 1.4008x; 1.3850x over previous
import functools

import jax
import jax.numpy as jnp
from jax import lax
from jax.experimental import pallas as pl
from jax.experimental.pallas import tpu as pltpu
from jax.experimental.pallas import tpu_sc as plsc

N = 1_000_000
D = 64
B = 98304
TWOB = 2 * B
NLAYER = 5

NC = 2
NS = 16
NW = NC * NS
PER_W = TWOB // NW
CHUNK = 128
NCHUNK = PER_W // CHUNK

BLK = 2048
NBLK = B // BLK


def _sc_gather_fn():
    mesh = plsc.VectorSubcoreMesh(core_axis_name="c", subcore_axis_name="s")

    @functools.partial(
        pl.kernel,
        out_type=jax.ShapeDtypeStruct((TWOB, D), jnp.float32),
        mesh=mesh,
        compiler_params=pltpu.CompilerParams(use_tc_tiling_on_sc=False),
        scratch_types=[
            pltpu.VMEM((NCHUNK, CHUNK), jnp.int32),
            pltpu.VMEM((CHUNK, D), jnp.float32),
            pltpu.VMEM((CHUNK, D), jnp.float32),
            pltpu.SemaphoreType.DMA,
            pltpu.SemaphoreType.DMA,
        ],
    )
    def sc_gather(u_hbm, table_hbm, out_hbm, idx_v, rows_a, rows_b, sem_a, sem_b):
        wid = lax.axis_index("s") * NC + lax.axis_index("c")
        rowbase = wid * PER_W
        pltpu.sync_copy(u_hbm.at[pl.ds(wid * NCHUNK, NCHUNK)], idx_v)

        def step(i, _):
            c0 = 2 * i
            cp_a = pltpu.async_copy(table_hbm.at[idx_v.at[c0]], rows_a, sem_a)
            cp_b = pltpu.async_copy(table_hbm.at[idx_v.at[c0 + 1]], rows_b, sem_b)
            cp_a.wait()
            pltpu.sync_copy(rows_a, out_hbm.at[pl.ds(rowbase + c0 * CHUNK, CHUNK)])
            cp_b.wait()
            pltpu.sync_copy(rows_b, out_hbm.at[pl.ds(rowbase + (c0 + 1) * CHUNK, CHUNK)])
            return 0

        lax.fori_loop(0, NCHUNK // 2, step, 0)

    return sc_gather


def _tc_loss_body(g_ref, lab_ref, lay_ref, w_ref, lt_ref, q_ref, acc_ref):
    blk = g_ref[...]
    ei = blk[:, :D]
    ej = blk[:, D:]
    x = jnp.dot(ei, w_ref[...], preferred_element_type=jnp.float32)
    y = jnp.dot(ej, w_ref[...], preferred_element_type=jnp.float32)
    rxy = jnp.sum(x * y, axis=1, keepdims=True)
    s1 = jnp.dot(x + y, lt_ref[...], preferred_element_type=jnp.float32)
    lay = lay_ref[...]
    onehot = (lay == lax.broadcasted_iota(jnp.int32, (BLK, 8), 1)).astype(jnp.float32)
    inner = rxy + jnp.sum(onehot * (s1 + q_ref[...]), axis=1, keepdims=True)
    t = lab_ref[...] * inner
    part = jnp.sum(jax.nn.log_sigmoid(t))

    @pl.when(pl.program_id(0) == 0)
    def _():
        acc_ref[0, 0] = 0.0

    acc_ref[0, 0] += -part


def kernel(u_i, u_j, this_layer, label, embedding, L_embedding, W):
    m = lax.iota(jnp.int32, TWOB)
    u_all = jnp.where(
        m % 2 == 0,
        jnp.repeat(u_i.astype(jnp.int32), 2),
        jnp.repeat(u_j.astype(jnp.int32), 2),
    ).reshape(TWOB // CHUNK, CHUNK)
    gathered = _sc_gather_fn()(u_all, embedding)
    g2 = gathered.reshape(B, 2 * D)

    lab = label.astype(jnp.float32).reshape(B, 1)
    lay = this_layer.astype(jnp.int32).reshape(B, 1)
    lt = jnp.zeros((D, 8), jnp.float32).at[:, :NLAYER].set(L_embedding.T)
    q = jnp.zeros((1, 8), jnp.float32).at[0, :NLAYER].set(
        jnp.sum(L_embedding * L_embedding, axis=1))

    loss = pl.pallas_call(
        _tc_loss_body,
        grid=(NBLK,),
        in_specs=[
            pl.BlockSpec((BLK, 2 * D), lambda i: (i, 0)),
            pl.BlockSpec((BLK, 1), lambda i: (i, 0)),
            pl.BlockSpec((BLK, 1), lambda i: (i, 0)),
            pl.BlockSpec((D, D), lambda i: (0, 0)),
            pl.BlockSpec((D, 8), lambda i: (0, 0)),
            pl.BlockSpec((1, 8), lambda i: (0, 0)),
        ],
        out_specs=pl.BlockSpec(memory_space=pltpu.SMEM),
        out_shape=jax.ShapeDtypeStruct((1, 1), jnp.float32),
    )(g2, lab, lay, W, lt, q)
    return loss[0, 0]
